# Initial kernel scaffold; baseline (speedup 1.0000x reference)
import jax
import jax.numpy as jnp
from jax.experimental import pallas as pl


def kernel(next_token_logits):
    raise NotImplementedError("write your pallas kernel here")


# speedup vs baseline: 63.8772x; 63.8772x over previous
import jax
import jax.numpy as jnp
from jax.experimental import pallas as pl
from jax.experimental.pallas import tpu as pltpu

VOCAB = 215680
ROWS = VOCAB // 128
W_ROW0 = 1184
W_LO = W_ROW0 * 128
WIN_ROWS = ROWS - W_ROW0
WIN = WIN_ROWS * 128
PAD_ROWS = 504
BAND_LO = 117
BAND_HI = 64117
EOS_LOC = 93
TOP_K = 50
CAP = 64
MIN_P = 0.05
TOP_P = 0.9
TEMPERATURE = 0.85
PAD_IDX = 0x7FFFFFFF
F32_TINY = float(jnp.finfo(jnp.float32).tiny)


def _rowvec(scalar, dtype):
    return jnp.zeros((1, 128), dtype) + scalar


def _threefry_bits(gi_u32):
    x0 = jnp.zeros_like(gi_u32)
    x1 = gi_u32
    ks0 = jnp.uint32(0)
    ks1 = jnp.uint32(1)
    ks2 = jnp.uint32(0x1BD11BDA) ^ ks0 ^ ks1
    rot1 = (13, 15, 26, 6)
    rot2 = (17, 29, 16, 24)
    x0 = x0 + ks0
    x1 = x1 + ks1
    ks = (ks1, ks2, ks0)
    for g in range(5):
        for r in (rot1 if g % 2 == 0 else rot2):
            x0 = x0 + x1
            x1 = (x1 << jnp.uint32(r)) | (x1 >> jnp.uint32(32 - r))
            x1 = x1 ^ x0
        x0 = x0 + ks[g % 3]
        x1 = x1 + ks[(g + 1) % 3] + jnp.uint32(g + 1)
    return x0 ^ x1


def _tc_body(x_ref, probs_ref, tok_ref,
             w_ref, key_ref, sv_ref, si_ref, osv_ref, osi_ref, mk_ref,
             loc_ref, cnt_ref):
    f32 = jnp.float32
    i32 = jnp.int32
    u32 = jnp.uint32
    neg_inf = f32(-jnp.inf)

    c = x_ref[0, W_ROW0:, :]
    u = x_ref[1, W_ROW0:, :]
    cfg = u + f32(2.0) * (c - u)
    ridx = jax.lax.broadcasted_iota(i32, (WIN_ROWS, 128), 0)
    lidx = jax.lax.broadcasted_iota(i32, (WIN_ROWS, 128), 1)
    flat = ridx * 128 + lidx
    valid = ((flat >= BAND_LO) & (flat < BAND_HI)) | (flat == EOS_LOC)
    w = jnp.where(valid, cfg, neg_inf)
    w_ref[0:WIN_ROWS, :] = w
    w_ref[WIN_ROWS:, :] = jnp.full((PAD_ROWS - WIN_ROWS, 128), neg_inf, f32)

    wall = w_ref[...] + f32(0.0)
    b = jax.lax.bitcast_convert_type(wall, u32)
    key = jnp.where((b >> u32(31)) == u32(1), ~b, b | u32(0x80000000))
    key_ref[...] = key

    def bit_step(i, t):
        cand = t | (u32(1) << (u32(31) - i.astype(u32)))
        cnt = jnp.sum((key_ref[...] >= cand).astype(i32))
        return jnp.where(cnt >= TOP_K, cand, t)

    t = jax.lax.fori_loop(0, 32, bit_step, u32(0))

    cnt_ref[0] = i32(0)
    sv_ref[...] = jnp.full((CAP, 128), neg_inf, f32)
    si_ref[...] = jnp.full((CAP, 128), PAD_IDX, i32)
    cfiota = (jax.lax.broadcasted_iota(i32, (8, 128), 0) * 128
              + jax.lax.broadcasted_iota(i32, (8, 128), 1))
    for j in range(PAD_ROWS // 8):
        kch = key_ref[8 * j:8 * j + 8, :]
        m0 = kch >= t
        c_j = jnp.sum(m0.astype(i32))
        base = cnt_ref[0]
        trip = jnp.minimum(c_j, i32(CAP) - base)

        @pl.when(trip > 0)
        def _():
            mk_ref[...] = jnp.where(m0, kch, u32(0))

            def ext(q, carry):
                alive = mk_ref[...] >= t
                pos = jnp.min(jnp.where(alive, cfiota, i32(2 ** 30)))
                r = pos // 128
                l = pos % 128
                lmask = jax.lax.broadcasted_iota(i32, (1, 128), 1) == l
                rowv = w_ref[pl.ds(8 * j + r, 1), :]
                val = jnp.sum(jnp.where(lmask, rowv, f32(0.0)))
                k = base + q
                sv_ref[pl.ds(k, 1), :] = _rowvec(val, f32)
                si_ref[pl.ds(k, 1), :] = _rowvec(8 * j * 128 + pos, i32)
                rowk = mk_ref[pl.ds(r, 1), :]
                mk_ref[pl.ds(r, 1), :] = jnp.where(lmask, u32(0), rowk)
                return carry

            jax.lax.fori_loop(0, trip, ext, i32(0))
            cnt_ref[0] = base + trip

    riota = jax.lax.broadcasted_iota(i32, (CAP, 128), 0)
    for k in range(CAP):
        sval = sv_ref[...]
        sidx = si_ref[...]
        mval = jnp.max(sval)
        midx = jnp.min(jnp.where(sval == mval, sidx, i32(PAD_IDX)))
        rstar = jnp.min(jnp.where((sval == mval) & (sidx == midx),
                                  riota, i32(2 ** 30)))
        osv_ref[k:k + 1, :] = _rowvec(mval, f32)
        osi_ref[k:k + 1, :] = _rowvec(midx, i32)
        loc_ref[k] = midx
        sv_ref[pl.ds(rstar, 1), :] = jnp.full((1, 128), neg_inf, f32)
        si_ref[pl.ds(rstar, 1), :] = jnp.full((1, 128), PAD_IDX, i32)

    osv = osv_ref[...]
    osi = osi_ref[...]
    m = jnp.max(osv)
    e = jnp.exp(osv - m)
    z1 = jnp.sum(e[:, 0:1])
    p = e / z1
    pmax = f32(1.0) / z1
    keep1 = p >= f32(MIN_P) * pmax
    v1 = jnp.where(keep1, osv, neg_inf)

    e2 = jnp.exp(v1 - m)
    z2 = jnp.sum(e2[:, 0:1])
    p2 = e2 / z2
    cs = p2
    d = 1
    while d < CAP:
        cs = cs + jnp.concatenate(
            [jnp.zeros((d, 128), f32), cs[:CAP - d]], axis=0)
        d *= 2
    csh = jnp.concatenate([jnp.zeros((1, 128), f32), cs[:CAP - 1]], axis=0)
    keep2 = csh <= f32(TOP_P)
    v2 = jnp.where(keep2, v1, neg_inf)

    s = v2 / f32(TEMPERATURE)
    m3 = m / f32(TEMPERATURE)
    e3 = jnp.exp(s - m3)
    z3 = jnp.sum(e3[:, 0:1])
    pf = e3 / z3

    gi = jnp.where(osi == i32(PAD_IDX), i32(0), i32(W_LO) + osi)
    bits = _threefry_bits(jax.lax.bitcast_convert_type(gi, u32))
    fb = (bits >> u32(9)) | u32(0x3F800000)
    frac = jax.lax.bitcast_convert_type(fb, f32) - f32(1.0)
    uu = jnp.maximum(f32(F32_TINY), frac + f32(F32_TINY))
    g = -jnp.log(-jnp.log(uu))
    score = s + g
    msc = jnp.max(score)
    tokv = jnp.min(jnp.where(score == msc, gi, i32(PAD_IDX)))
    tok_ref[0, 0] = tokv

    probs_ref[...] = jnp.zeros((ROWS, 128), f32)
    n = cnt_ref[0]
    liota = jax.lax.broadcasted_iota(i32, (1, 128), 1)
    for k in range(CAP):
        @pl.when(k < n)
        def _():
            loc = loc_ref[k]
            r = W_ROW0 + loc // 128
            l = loc % 128
            valr = pf[k:k + 1, :]
            rowv = probs_ref[pl.ds(r, 1), :]
            probs_ref[pl.ds(r, 1), :] = jnp.where(liota == l, valr, rowv)


def _run_tc(x3, interpret=False):
    return pl.pallas_call(
        _tc_body,
        out_shape=(jax.ShapeDtypeStruct((ROWS, 128), jnp.float32),
                   jax.ShapeDtypeStruct((1, 1), jnp.int32)),
        in_specs=[pl.BlockSpec(memory_space=pltpu.VMEM)],
        out_specs=(pl.BlockSpec(memory_space=pltpu.VMEM),
                   pl.BlockSpec(memory_space=pltpu.SMEM)),
        scratch_shapes=[
            pltpu.VMEM((PAD_ROWS, 128), jnp.float32),
            pltpu.VMEM((PAD_ROWS, 128), jnp.uint32),
            pltpu.VMEM((CAP, 128), jnp.float32),
            pltpu.VMEM((CAP, 128), jnp.int32),
            pltpu.VMEM((CAP, 128), jnp.float32),
            pltpu.VMEM((CAP, 128), jnp.int32),
            pltpu.VMEM((8, 128), jnp.uint32),
            pltpu.SMEM((CAP,), jnp.int32),
            pltpu.SMEM((1,), jnp.int32),
        ],
        interpret=interpret,
    )(x3)


def kernel(next_token_logits):
    x3 = next_token_logits.reshape(2, ROWS, 128)
    probs, tok = _run_tc(x3)
    return probs.reshape(1, VOCAB), tok.reshape(1)


# speedup vs baseline: 76.2212x; 1.1932x over previous
import jax
import jax.numpy as jnp
from jax.experimental import pallas as pl
from jax.experimental.pallas import tpu as pltpu
from jax.experimental.pallas import tpu_sc as plsc

VOCAB = 215680
ROWS = VOCAB // 128
W_ROW0 = 1184
W_LO = W_ROW0 * 128
WIN_ROWS = ROWS - W_ROW0
WIN = WIN_ROWS * 128
PAD_ROWS = 504
BAND_LO = 117
BAND_HI = 64117
EOS_LOC = 93
TOP_K = 50
CAP = 64
MIN_P = 0.05
TOP_P = 0.9
TEMPERATURE = 0.85
PAD_IDX = 0x7FFFFFFF
PAD_BASE = 0x40000000
F32_TINY = float(jnp.finfo(jnp.float32).tiny)


def _rowvec(scalar, dtype):
    return jnp.zeros((1, 128), dtype) + scalar


def _threefry_bits(gi_u32):
    x0 = jnp.zeros_like(gi_u32)
    x1 = gi_u32
    ks0 = jnp.uint32(0)
    ks1 = jnp.uint32(1)
    ks2 = jnp.uint32(0x1BD11BDA) ^ ks0 ^ ks1
    rot1 = (13, 15, 26, 6)
    rot2 = (17, 29, 16, 24)
    x0 = x0 + ks0
    x1 = x1 + ks1
    ks = (ks1, ks2, ks0)
    for g in range(5):
        for r in (rot1 if g % 2 == 0 else rot2):
            x0 = x0 + x1
            x1 = (x1 << jnp.uint32(r)) | (x1 >> jnp.uint32(32 - r))
            x1 = x1 ^ x0
        x0 = x0 + ks[g % 3]
        x1 = x1 + ks[(g + 1) % 3] + jnp.uint32(g + 1)
    return x0 ^ x1


def _tc_body(x_ref, pk_ref, ik_ref, tok_ref,
             w_ref, key_ref, sv_ref, si_ref, mk_ref, cnt_ref):
    f32 = jnp.float32
    i32 = jnp.int32
    u32 = jnp.uint32
    neg_inf = f32(-jnp.inf)

    c = x_ref[0, W_ROW0:, :]
    u = x_ref[1, W_ROW0:, :]
    cfg = u + f32(2.0) * (c - u)
    ridx = jax.lax.broadcasted_iota(i32, (WIN_ROWS, 128), 0)
    lidx = jax.lax.broadcasted_iota(i32, (WIN_ROWS, 128), 1)
    flat = ridx * 128 + lidx
    valid = ((flat >= BAND_LO) & (flat < BAND_HI)) | (flat == EOS_LOC)
    w = jnp.where(valid, cfg, neg_inf)
    w_ref[0:WIN_ROWS, :] = w
    w_ref[WIN_ROWS:, :] = jnp.full((PAD_ROWS - WIN_ROWS, 128), neg_inf, f32)

    wall = w_ref[...] + f32(0.0)
    b = jax.lax.bitcast_convert_type(wall, u32)
    key = jnp.where((b >> u32(31)) == u32(1), ~b, b | u32(0x80000000))
    key_ref[...] = key

    def bit_step(i, t):
        cand = t | (u32(1) << (u32(31) - i.astype(u32)))
        cnt = jnp.sum((key_ref[...] >= cand).astype(i32))
        return jnp.where(cnt >= TOP_K, cand, t)

    t = jax.lax.fori_loop(0, 32, bit_step, u32(0))

    cnt_ref[0] = i32(0)
    sv_ref[...] = jnp.full((CAP, 128), neg_inf, f32)
    si_ref[...] = (jnp.full((CAP, 128), PAD_BASE, i32)
                   + jax.lax.broadcasted_iota(i32, (CAP, 128), 0))
    cfiota = (jax.lax.broadcasted_iota(i32, (8, 128), 0) * 128
              + jax.lax.broadcasted_iota(i32, (8, 128), 1))
    for j in range(PAD_ROWS // 8):
        kch = key_ref[8 * j:8 * j + 8, :]
        m0 = kch >= t
        c_j = jnp.sum(m0.astype(i32))
        base = cnt_ref[0]
        trip = jnp.minimum(c_j, i32(CAP) - base)

        @pl.when(trip > 0)
        def _():
            mk_ref[...] = jnp.where(m0, kch, u32(0))

            def ext(q, carry):
                alive = mk_ref[...] >= t
                pos = jnp.min(jnp.where(alive, cfiota, i32(2 ** 30)))
                r = pos // 128
                l = pos % 128
                lmask = jax.lax.broadcasted_iota(i32, (1, 128), 1) == l
                rowv = w_ref[pl.ds(8 * j + r, 1), :]
                val = jnp.sum(jnp.where(lmask, rowv, f32(0.0)))
                k = base + q
                sv_ref[pl.ds(k, 1), :] = _rowvec(val, f32)
                si_ref[pl.ds(k, 1), :] = _rowvec(8 * j * 128 + pos, i32)
                rowk = mk_ref[pl.ds(r, 1), :]
                mk_ref[pl.ds(r, 1), :] = jnp.where(lmask, u32(0), rowk)
                return carry

            jax.lax.fori_loop(0, trip, ext, i32(0))
            cnt_ref[0] = base + trip

    riota = jax.lax.broadcasted_iota(i32, (CAP, 128), 0)
    osv = sv_ref[...]
    osi = si_ref[...]
    k2 = 2
    while k2 <= CAP:
        dirdesc = (riota & k2) == 0
        j = k2 // 2
        while j >= 1:
            lobit = (riota & j) == 0
            svp = jnp.where(lobit,
                            jnp.concatenate([osv[j:], osv[:j]], axis=0),
                            jnp.concatenate([osv[CAP - j:], osv[:CAP - j]],
                                            axis=0))
            sip = jnp.where(lobit,
                            jnp.concatenate([osi[j:], osi[:j]], axis=0),
                            jnp.concatenate([osi[CAP - j:], osi[:CAP - j]],
                                            axis=0))
            afirst = (osv > svp) | ((osv == svp) & (osi < sip))
            take_self = ((afirst == lobit) == dirdesc)
            osv = jnp.where(take_self, osv, svp)
            osi = jnp.where(take_self, osi, sip)
            j //= 2
        k2 *= 2

    m = jnp.max(osv)
    e = jnp.exp(osv - m)
    z1 = jnp.sum(e[:, 0:1])
    p = e / z1
    pmax = f32(1.0) / z1
    keep1 = p >= f32(MIN_P) * pmax
    v1 = jnp.where(keep1, osv, neg_inf)

    e2 = jnp.exp(v1 - m)
    z2 = jnp.sum(e2[:, 0:1])
    p2 = e2 / z2
    cs = p2
    d = 1
    while d < CAP:
        cs = cs + jnp.concatenate(
            [jnp.zeros((d, 128), f32), cs[:CAP - d]], axis=0)
        d *= 2
    csh = jnp.concatenate([jnp.zeros((1, 128), f32), cs[:CAP - 1]], axis=0)
    keep2 = csh <= f32(TOP_P)
    v2 = jnp.where(keep2, v1, neg_inf)

    s = v2 / f32(TEMPERATURE)
    m3 = m / f32(TEMPERATURE)
    e3 = jnp.exp(s - m3)
    z3 = jnp.sum(e3[:, 0:1])
    pf = e3 / z3

    gi = jnp.where(osi >= i32(PAD_BASE), i32(0), i32(W_LO) + osi)
    bits = _threefry_bits(jax.lax.bitcast_convert_type(gi, u32))
    fb = (bits >> u32(9)) | u32(0x3F800000)
    frac = jax.lax.bitcast_convert_type(fb, f32) - f32(1.0)
    uu = jnp.maximum(f32(F32_TINY), frac + f32(F32_TINY))
    g = -jnp.log(-jnp.log(uu))
    score = s + g
    msc = jnp.max(score)
    tokv = jnp.min(jnp.where(score == msc, gi, i32(PAD_IDX)))
    tok_ref[0, 0] = tokv

    liota = jax.lax.broadcasted_iota(i32, (1, 128), 1)
    pk = jnp.zeros((1, 128), f32)
    ik = jnp.full((1, 128), PAD_BASE, i32)
    gvi = jnp.where(osi >= i32(PAD_BASE), osi, i32(W_LO) + osi)
    for k in range(CAP):
        pk = jnp.where(liota == k, pf[k:k + 1, :], pk)
        ik = jnp.where(liota == k, gvi[k:k + 1, :], ik)
    pk_ref[...] = pk
    ik_ref[...] = ik


def _run_tc(x3, interpret=False):
    return pl.pallas_call(
        _tc_body,
        out_shape=(jax.ShapeDtypeStruct((1, 128), jnp.float32),
                   jax.ShapeDtypeStruct((1, 128), jnp.int32),
                   jax.ShapeDtypeStruct((1, 1), jnp.int32)),
        in_specs=[pl.BlockSpec(memory_space=pltpu.VMEM)],
        out_specs=(pl.BlockSpec(memory_space=pltpu.VMEM),
                   pl.BlockSpec(memory_space=pltpu.VMEM),
                   pl.BlockSpec(memory_space=pltpu.SMEM)),
        scratch_shapes=[
            pltpu.VMEM((PAD_ROWS, 128), jnp.float32),
            pltpu.VMEM((PAD_ROWS, 128), jnp.uint32),
            pltpu.VMEM((CAP, 128), jnp.float32),
            pltpu.VMEM((CAP, 128), jnp.int32),
            pltpu.VMEM((8, 128), jnp.uint32),
            pltpu.SMEM((1,), jnp.int32),
        ],
        interpret=interpret,
    )(x3)


SC_NW = 32
SC_CHUNK = 6784
SC_LAST = VOCAB - (SC_NW - 1) * SC_CHUNK


def _sc_body(pk_hbm, ik_hbm, out_hbm, pv_ref, iv_ref, zbuf_ref):
    wid = jax.lax.axis_index("s") * 2 + jax.lax.axis_index("c")
    pltpu.sync_copy(pk_hbm, pv_ref)
    pltpu.sync_copy(ik_hbm, iv_ref)
    lo = wid * SC_CHUNK

    def assemble(nwords):
        def zstep(i, carry):
            zbuf_ref[pl.ds(i * 16, 16)] = jnp.zeros((16,), jnp.float32)
            return carry

        jax.lax.fori_loop(0, nwords // 16, zstep, 0)
        for jj in range(8):
            ivv = iv_ref[pl.ds(jj * 16, 16)]
            pvv = pv_ref[pl.ds(jj * 16, 16)]
            msk = (ivv >= lo) & (ivv < lo + nwords)
            plsc.store_scatter(zbuf_ref, [ivv - lo], pvv, mask=msk)
        pltpu.sync_copy(zbuf_ref.at[pl.ds(0, nwords)],
                        out_hbm.at[pl.ds(lo, nwords)])

    @pl.when(wid < SC_NW - 1)
    def _():
        assemble(SC_CHUNK)

    @pl.when(wid == SC_NW - 1)
    def _():
        assemble(SC_LAST)


def _run_sc(pk_flat, ik_flat):
    return pl.kernel(
        _sc_body,
        mesh=plsc.VectorSubcoreMesh(core_axis_name="c", subcore_axis_name="s"),
        out_type=jax.ShapeDtypeStruct((VOCAB,), jnp.float32),
        scratch_types=[
            pltpu.VMEM((128,), jnp.float32),
            pltpu.VMEM((128,), jnp.int32),
            pltpu.VMEM((SC_CHUNK,), jnp.float32),
        ],
        compiler_params=pltpu.CompilerParams(needs_layout_passes=False),
    )(pk_flat, ik_flat)


def kernel(next_token_logits):
    x3 = next_token_logits.reshape(2, ROWS, 128)
    pk, ik, tok = _run_tc(x3)
    probs = _run_sc(pk.reshape(128), ik.reshape(128))
    return probs.reshape(1, VOCAB), tok.reshape(1)


# speedup vs baseline: 108.8212x; 1.4277x over previous
import jax
import jax.numpy as jnp
from jax.experimental import pallas as pl
from jax.experimental.pallas import tpu as pltpu
from jax.experimental.pallas import tpu_sc as plsc

VOCAB = 215680
ROWS = VOCAB // 128
W_ROW0 = 1184
W_LO = W_ROW0 * 128
WIN_ROWS = ROWS - W_ROW0
PAD_ROWS = 504
BAND_LO = 117
BAND_HI = 64117
EOS_LOC = 93
TOP_K = 50
CAP = 64
NR = 16
MIN_P = 0.05
TOP_P = 0.9
TEMPERATURE = 0.85
PAD_IDX = 0x7FFFFFFF
PAD_BASE = 0x40000000
F32_TINY = float(jnp.finfo(jnp.float32).tiny)


def _threefry_bits(gi_u32):
    x0 = jnp.zeros_like(gi_u32)
    x1 = gi_u32
    ks0 = jnp.uint32(0)
    ks1 = jnp.uint32(1)
    ks2 = jnp.uint32(0x1BD11BDA) ^ ks0 ^ ks1
    rot1 = (13, 15, 26, 6)
    rot2 = (17, 29, 16, 24)
    x0 = x0 + ks0
    x1 = x1 + ks1
    ks = (ks1, ks2, ks0)
    for g in range(5):
        for r in (rot1 if g % 2 == 0 else rot2):
            x0 = x0 + x1
            x1 = (x1 << jnp.uint32(r)) | (x1 >> jnp.uint32(32 - r))
            x1 = x1 ^ x0
        x0 = x0 + ks[g % 3]
        x1 = x1 + ks[(g + 1) % 3] + jnp.uint32(g + 1)
    return x0 ^ x1


def _tc_body(x_ref, pk_ref, ik_ref, tok_ref, key_ref):
    f32 = jnp.float32
    i32 = jnp.int32
    u32 = jnp.uint32
    neg_inf = f32(-jnp.inf)

    c = x_ref[0, W_ROW0:, :]
    u = x_ref[1, W_ROW0:, :]
    cfg = u + f32(2.0) * (c - u)
    ridx = jax.lax.broadcasted_iota(i32, (WIN_ROWS, 128), 0)
    lidx = jax.lax.broadcasted_iota(i32, (WIN_ROWS, 128), 1)
    flat = ridx * 128 + lidx
    valid = ((flat >= BAND_LO) & (flat < BAND_HI)) | (flat == EOS_LOC)
    w = jnp.where(valid, cfg + f32(0.0), neg_inf)
    b = jax.lax.bitcast_convert_type(w, u32)
    key = jnp.where((b >> u32(31)) == u32(1), ~b, b | u32(0x80000000))
    key_ref[0:WIN_ROWS, :] = key
    key_ref[WIN_ROWS:, :] = jnp.zeros((PAD_ROWS - WIN_ROWS, 128), u32)

    def bit_step(i, t):
        sh = u32(30) - u32(2) * i.astype(u32)
        b1 = u32(2) << sh
        b2 = u32(1) << sh
        ka = key_ref[...]
        c1 = jnp.sum((ka >= (t | b1)).astype(i32))
        c2 = jnp.sum((ka >= (t | b2)).astype(i32))
        c3 = jnp.sum((ka >= (t | b1 | b2)).astype(i32))
        hi_ok = c1 >= TOP_K
        t2 = jnp.where(hi_ok, t | b1, t)
        lo_cnt = jnp.where(hi_ok, c3, c2)
        return jnp.where(lo_cnt >= TOP_K, t2 | b2, t2)

    t = jax.lax.fori_loop(0, 16, bit_step, u32(0))

    riota504 = jax.lax.broadcasted_iota(i32, (PAD_ROWS, 128), 0)
    lane1 = jax.lax.broadcasted_iota(i32, (1, 128), 1)
    pks = []
    pis = []
    for r in range(NR):
        ka = key_ref[...]
        alive = ka >= t
        rowidx = jnp.min(jnp.where(alive, riota504, i32(1 << 20)),
                         axis=0, keepdims=True)
        eq = (riota504 == rowidx) & alive
        ka_i = jax.lax.bitcast_convert_type(ka, i32)
        kk = jax.lax.bitcast_convert_type(
            jnp.sum(jnp.where(eq, ka_i, i32(0)), axis=0, keepdims=True), u32)
        live = kk > u32(0)
        pks.append(kk)
        pis.append(jnp.where(live, rowidx * 128 + lane1,
                             i32(PAD_BASE) + r * 128 + lane1))
        key_ref[...] = jnp.where(eq, u32(0), ka)
    psk = jnp.concatenate(pks, axis=0)
    psi = jnp.concatenate(pis, axis=0)

    live = psk > u32(0)
    vb = jnp.where(psk >= u32(0x80000000), psk ^ u32(0x80000000), ~psk)
    psv = jnp.where(live, jax.lax.bitcast_convert_type(vb, f32), neg_inf)

    riota = jax.lax.broadcasted_iota(i32, (NR, 128), 0)
    liota = jax.lax.broadcasted_iota(i32, (NR, 128), 1)
    fr = liota * NR + riota
    N = NR * 128

    def roll_rows(x, d):
        return jnp.concatenate([x[d:], x[:d]], axis=0)

    k2 = 2
    while k2 <= N:
        dirdesc = (fr & k2) == 0
        j = k2 // 2
        while j >= 1:
            lobit = (fr & j) == 0
            if j < NR:
                up_v, dn_v = roll_rows(psv, j), roll_rows(psv, NR - j)
                up_i, dn_i = roll_rows(psi, j), roll_rows(psi, NR - j)
            else:
                d = j // NR
                up_v = pltpu.roll(psv, 128 - d, axis=1)
                dn_v = pltpu.roll(psv, d, axis=1)
                up_i = pltpu.roll(psi, 128 - d, axis=1)
                dn_i = pltpu.roll(psi, d, axis=1)
            svp = jnp.where(lobit, up_v, dn_v)
            sip = jnp.where(lobit, up_i, dn_i)
            afirst = (psv > svp) | ((psv == svp) & (psi < sip))
            take_self = ((afirst == lobit) == dirdesc)
            psv = jnp.where(take_self, psv, svp)
            psi = jnp.where(take_self, psi, sip)
            j //= 2
        k2 *= 2

    keepcap = fr < CAP
    psv = jnp.where(keepcap, psv, neg_inf)
    psi = jnp.where(keepcap, psi, i32(PAD_BASE) + fr)

    m = jnp.max(psv)
    e = jnp.exp(psv - m)
    z1 = jnp.sum(e)
    p = e / z1
    pmax = f32(1.0) / z1
    keep1 = p >= f32(MIN_P) * pmax
    v1 = jnp.where(keep1, psv, neg_inf)

    e2 = jnp.exp(v1 - m)
    z2 = jnp.sum(e2)
    p2 = e2 / z2
    cs = p2
    d = 1
    while d < NR:
        cs = cs + jnp.concatenate(
            [jnp.zeros((d, 128), f32), cs[:NR - d]], axis=0)
        d *= 2
    lane_tot = cs[NR - 1:NR, :]
    ex = pltpu.roll(lane_tot, 1, axis=1)
    ex = jnp.where(lane1 >= 1, ex, f32(0.0))
    d = 1
    while d < 128:
        sh = pltpu.roll(ex, d, axis=1)
        ex = ex + jnp.where(lane1 >= d, sh, f32(0.0))
        d *= 2
    ci = cs + ex
    top = pltpu.roll(ci[NR - 1:NR, :], 1, axis=1)
    top = jnp.where(lane1 >= 1, top, f32(0.0))
    csh = jnp.concatenate([top, ci[:NR - 1]], axis=0)
    keep2 = csh <= f32(TOP_P)
    v2 = jnp.where(keep2, v1, neg_inf)

    s = v2 / f32(TEMPERATURE)
    m3 = m / f32(TEMPERATURE)
    e3 = jnp.exp(s - m3)
    z3 = jnp.sum(e3)
    pf = e3 / z3

    gi = jnp.where(psi >= i32(PAD_BASE), i32(0), i32(W_LO) + psi)
    bits = _threefry_bits(jax.lax.bitcast_convert_type(gi, u32))
    fb = (bits >> u32(9)) | u32(0x3F800000)
    frac = jax.lax.bitcast_convert_type(fb, f32) - f32(1.0)
    uu = jnp.maximum(f32(F32_TINY), frac + f32(F32_TINY))
    g = -jnp.log(-jnp.log(uu))
    score = s + g
    msc = jnp.max(score)
    tokv = jnp.min(jnp.where(score == msc, gi, i32(PAD_IDX)))
    tok_ref[0, 0] = tokv

    ncols = CAP // NR
    gvi = jnp.where(psi >= i32(PAD_BASE), i32(PAD_IDX), i32(W_LO) + psi)
    pk_ref[...] = pf[:, 0:ncols]
    ik_ref[...] = gvi[:, 0:ncols]


def _run_tc(x3, interpret=False):
    return pl.pallas_call(
        _tc_body,
        out_shape=(jax.ShapeDtypeStruct((NR, CAP // NR), jnp.float32),
                   jax.ShapeDtypeStruct((NR, CAP // NR), jnp.int32),
                   jax.ShapeDtypeStruct((1, 1), jnp.int32)),
        in_specs=[pl.BlockSpec(memory_space=pltpu.VMEM)],
        out_specs=(pl.BlockSpec(memory_space=pltpu.VMEM),
                   pl.BlockSpec(memory_space=pltpu.VMEM),
                   pl.BlockSpec(memory_space=pltpu.SMEM)),
        scratch_shapes=[
            pltpu.VMEM((PAD_ROWS, 128), jnp.uint32),
        ],
        interpret=interpret,
    )(x3)


SC_NW = 32
SC_CHUNK = 6784
SC_LAST = VOCAB - (SC_NW - 1) * SC_CHUNK


def _sc_body(pk_hbm, ik_hbm, out_hbm, pv_ref, iv_ref, zbuf_ref):
    wid = jax.lax.axis_index("s") * 2 + jax.lax.axis_index("c")
    pltpu.sync_copy(pk_hbm, pv_ref)
    pltpu.sync_copy(ik_hbm, iv_ref)
    lo = wid * SC_CHUNK

    def assemble(nwords):
        def zstep(i, carry):
            zbuf_ref[pl.ds(i * 16, 16)] = jnp.zeros((16,), jnp.float32)
            return carry

        jax.lax.fori_loop(0, nwords // 16, zstep, 0)
        for jj in range(CAP // 16):
            ivv = iv_ref[pl.ds(jj * 16, 16)]
            pvv = pv_ref[pl.ds(jj * 16, 16)]
            msk = (ivv >= lo) & (ivv < lo + nwords)
            plsc.store_scatter(zbuf_ref, [ivv - lo], pvv, mask=msk)
        pltpu.sync_copy(zbuf_ref.at[pl.ds(0, nwords)],
                        out_hbm.at[pl.ds(lo, nwords)])

    @pl.when(wid < SC_NW - 1)
    def _():
        assemble(SC_CHUNK)

    @pl.when(wid == SC_NW - 1)
    def _():
        assemble(SC_LAST)


def _run_sc(pk_flat, ik_flat):
    return pl.kernel(
        _sc_body,
        mesh=plsc.VectorSubcoreMesh(core_axis_name="c", subcore_axis_name="s"),
        out_type=jax.ShapeDtypeStruct((VOCAB,), jnp.float32),
        scratch_types=[
            pltpu.VMEM((CAP,), jnp.float32),
            pltpu.VMEM((CAP,), jnp.int32),
            pltpu.VMEM((SC_CHUNK,), jnp.float32),
        ],
        compiler_params=pltpu.CompilerParams(needs_layout_passes=False),
    )(pk_flat, ik_flat)


def kernel(next_token_logits):
    x3 = next_token_logits.reshape(2, ROWS, 128)
    pk, ik, tok = _run_tc(x3)
    probs = _run_sc(pk.reshape(CAP), ik.reshape(CAP))
    return probs.reshape(1, VOCAB), tok.reshape(1)


# speedup vs baseline: 111.5596x; 1.0252x over previous
import jax
import jax.numpy as jnp
from jax.experimental import pallas as pl
from jax.experimental.pallas import tpu as pltpu
from jax.experimental.pallas import tpu_sc as plsc

VOCAB = 215680
ROWS = VOCAB // 128
W_ROW0 = 1184
W_LO = W_ROW0 * 128
WIN_ROWS = ROWS - W_ROW0
PAD_ROWS = 504
BAND_LO = 117
BAND_HI = 64117
EOS_LOC = 93
TOP_K = 50
CAP = 64
NR = 16
MIN_P = 0.05
TOP_P = 0.9
TEMPERATURE = 0.85
PAD_IDX = 0x7FFFFFFF
PAD_BASE = 0x40000000
F32_TINY = float(jnp.finfo(jnp.float32).tiny)


def _threefry_bits(gi_u32):
    x0 = jnp.zeros_like(gi_u32)
    x1 = gi_u32
    ks0 = jnp.uint32(0)
    ks1 = jnp.uint32(1)
    ks2 = jnp.uint32(0x1BD11BDA) ^ ks0 ^ ks1
    rot1 = (13, 15, 26, 6)
    rot2 = (17, 29, 16, 24)
    x0 = x0 + ks0
    x1 = x1 + ks1
    ks = (ks1, ks2, ks0)
    for g in range(5):
        for r in (rot1 if g % 2 == 0 else rot2):
            x0 = x0 + x1
            x1 = (x1 << jnp.uint32(r)) | (x1 >> jnp.uint32(32 - r))
            x1 = x1 ^ x0
        x0 = x0 + ks[g % 3]
        x1 = x1 + ks[(g + 1) % 3] + jnp.uint32(g + 1)
    return x0 ^ x1


def _tc_body(x_ref, pk_ref, ik_ref, tok_ref, key_ref):
    f32 = jnp.float32
    i32 = jnp.int32
    u32 = jnp.uint32
    neg_inf = f32(-jnp.inf)

    c = x_ref[0, W_ROW0:, :]
    u = x_ref[1, W_ROW0:, :]
    cfg = u + f32(2.0) * (c - u)
    ridx = jax.lax.broadcasted_iota(i32, (WIN_ROWS, 128), 0)
    lidx = jax.lax.broadcasted_iota(i32, (WIN_ROWS, 128), 1)
    flat = ridx * 128 + lidx
    valid = ((flat >= BAND_LO) & (flat < BAND_HI)) | (flat == EOS_LOC)
    w = jnp.where(valid, cfg + f32(0.0), neg_inf)
    b = jax.lax.bitcast_convert_type(w, u32)
    key = jnp.where((b >> u32(31)) == u32(1), ~b, b | u32(0x80000000))
    key_ref[0:WIN_ROWS, :] = key
    key_ref[WIN_ROWS:, :] = jnp.zeros((PAD_ROWS - WIN_ROWS, 128), u32)

    def bit_step(i, t):
        sh = u32(30) - u32(2) * i.astype(u32)
        b1 = u32(2) << sh
        b2 = u32(1) << sh
        ka = key_ref[...]
        c1 = jnp.sum((ka >= (t | b1)).astype(i32))
        c2 = jnp.sum((ka >= (t | b2)).astype(i32))
        c3 = jnp.sum((ka >= (t | b1 | b2)).astype(i32))
        hi_ok = c1 >= TOP_K
        t2 = jnp.where(hi_ok, t | b1, t)
        lo_cnt = jnp.where(hi_ok, c3, c2)
        return jnp.where(lo_cnt >= TOP_K, t2 | b2, t2)

    t = jax.lax.fori_loop(0, 16, bit_step, u32(0))

    riota504 = jax.lax.broadcasted_iota(i32, (PAD_ROWS, 128), 0)
    lane1 = jax.lax.broadcasted_iota(i32, (1, 128), 1)
    pks = []
    pis = []
    for r in range(NR):
        ka = key_ref[...]
        alive = ka >= t
        rowidx = jnp.min(jnp.where(alive, riota504, i32(1 << 20)),
                         axis=0, keepdims=True)
        eq = (riota504 == rowidx) & alive
        ka_i = jax.lax.bitcast_convert_type(ka, i32)
        kk = jax.lax.bitcast_convert_type(
            jnp.sum(jnp.where(eq, ka_i, i32(0)), axis=0, keepdims=True), u32)
        live = kk > u32(0)
        pks.append(kk)
        pis.append(jnp.where(live, rowidx * 128 + lane1,
                             i32(PAD_BASE) + r * 128 + lane1))
        key_ref[...] = jnp.where(eq, u32(0), ka)
    psk = jnp.concatenate(pks, axis=0)
    psi = jnp.concatenate(pis, axis=0)

    live = psk > u32(0)
    vb = jnp.where(psk >= u32(0x80000000), psk ^ u32(0x80000000), ~psk)
    psv = jnp.where(live, jax.lax.bitcast_convert_type(vb, f32), neg_inf)

    riota = jax.lax.broadcasted_iota(i32, (NR, 128), 0)
    liota = jax.lax.broadcasted_iota(i32, (NR, 128), 1)
    fr = liota * NR + riota
    N = NR * 128

    def roll_rows(x, d):
        return jnp.concatenate([x[d:], x[:d]], axis=0)

    k2 = 2
    while k2 <= N:
        dirdesc = (fr & k2) == 0
        j = k2 // 2
        while j >= 1:
            lobit = (fr & j) == 0
            if j < NR:
                up_v, dn_v = roll_rows(psv, j), roll_rows(psv, NR - j)
                up_i, dn_i = roll_rows(psi, j), roll_rows(psi, NR - j)
            else:
                d = j // NR
                up_v = pltpu.roll(psv, 128 - d, axis=1)
                dn_v = pltpu.roll(psv, d, axis=1)
                up_i = pltpu.roll(psi, 128 - d, axis=1)
                dn_i = pltpu.roll(psi, d, axis=1)
            svp = jnp.where(lobit, up_v, dn_v)
            sip = jnp.where(lobit, up_i, dn_i)
            afirst = (psv > svp) | ((psv == svp) & (psi < sip))
            take_self = ((afirst == lobit) == dirdesc)
            psv = jnp.where(take_self, psv, svp)
            psi = jnp.where(take_self, psi, sip)
            j //= 2
        k2 *= 2

    keepcap = fr < CAP
    psv = jnp.where(keepcap, psv, neg_inf)
    psi = jnp.where(keepcap, psi, i32(PAD_BASE) + fr)

    m = jnp.max(psv)
    e = jnp.exp(psv - m)
    z1 = jnp.sum(e)
    p = e / z1
    pmax = f32(1.0) / z1
    keep1 = p >= f32(MIN_P) * pmax
    v1 = jnp.where(keep1, psv, neg_inf)

    e2 = jnp.exp(v1 - m)
    z2 = jnp.sum(e2)
    p2 = e2 / z2
    cs = p2
    d = 1
    while d < NR:
        cs = cs + jnp.concatenate(
            [jnp.zeros((d, 128), f32), cs[:NR - d]], axis=0)
        d *= 2
    lane_tot = cs[NR - 1:NR, :]
    ex = pltpu.roll(lane_tot, 1, axis=1)
    ex = jnp.where(lane1 >= 1, ex, f32(0.0))
    d = 1
    while d < 128:
        sh = pltpu.roll(ex, d, axis=1)
        ex = ex + jnp.where(lane1 >= d, sh, f32(0.0))
        d *= 2
    ci = cs + ex
    top = pltpu.roll(ci[NR - 1:NR, :], 1, axis=1)
    top = jnp.where(lane1 >= 1, top, f32(0.0))
    csh = jnp.concatenate([top, ci[:NR - 1]], axis=0)
    keep2 = csh <= f32(TOP_P)
    v2 = jnp.where(keep2, v1, neg_inf)

    s = v2 / f32(TEMPERATURE)
    m3 = m / f32(TEMPERATURE)
    e3 = jnp.exp(s - m3)
    z3 = jnp.sum(e3)
    pf = e3 / z3

    gi = jnp.where(psi >= i32(PAD_BASE), i32(0), i32(W_LO) + psi)
    bits = _threefry_bits(jax.lax.bitcast_convert_type(gi, u32))
    fb = (bits >> u32(9)) | u32(0x3F800000)
    frac = jax.lax.bitcast_convert_type(fb, f32) - f32(1.0)
    uu = jnp.maximum(f32(F32_TINY), frac + f32(F32_TINY))
    g = -jnp.log(-jnp.log(uu))
    score = s + g
    msc = jnp.max(score)
    tokv = jnp.min(jnp.where(score == msc, gi, i32(PAD_IDX)))
    tok_ref[0, 0] = tokv

    ncols = CAP // NR
    gvi = jnp.where(psi >= i32(PAD_BASE), i32(PAD_IDX), i32(W_LO) + psi)
    pk_ref[...] = pf[:, 0:ncols]
    ik_ref[...] = gvi[:, 0:ncols]


def _run_tc(x3, interpret=False):
    return pl.pallas_call(
        _tc_body,
        out_shape=(jax.ShapeDtypeStruct((NR, CAP // NR), jnp.float32),
                   jax.ShapeDtypeStruct((NR, CAP // NR), jnp.int32),
                   jax.ShapeDtypeStruct((1, 1), jnp.int32)),
        in_specs=[pl.BlockSpec(memory_space=pltpu.VMEM)],
        out_specs=(pl.BlockSpec(memory_space=pltpu.VMEM),
                   pl.BlockSpec(memory_space=pltpu.VMEM),
                   pl.BlockSpec(memory_space=pltpu.SMEM)),
        scratch_shapes=[
            pltpu.VMEM((PAD_ROWS, 128), jnp.uint32),
        ],
        interpret=interpret,
    )(x3)


SC_NW = 32
SC_CHUNK = 6784
SC_LAST = VOCAB - (SC_NW - 1) * SC_CHUNK


def _sc_body(pk_hbm, ik_hbm, out_hbm, pv_ref, iv_ref, zbuf_ref):
    wid = jax.lax.axis_index("s") * 2 + jax.lax.axis_index("c")
    pltpu.sync_copy(pk_hbm, pv_ref)
    pltpu.sync_copy(ik_hbm, iv_ref)
    lo = wid * SC_CHUNK

    def assemble(nwords):
        def zstep(i, carry):
            for q in range(8):
                zbuf_ref[pl.ds(i * 128 + q * 16, 16)] = (
                    jnp.zeros((16,), jnp.float32))
            return carry

        jax.lax.fori_loop(0, nwords // 128, zstep, 0)
        for jj in range(CAP // 16):
            ivv = iv_ref[pl.ds(jj * 16, 16)]
            pvv = pv_ref[pl.ds(jj * 16, 16)]
            msk = (ivv >= lo) & (ivv < lo + nwords)
            plsc.store_scatter(zbuf_ref, [ivv - lo], pvv, mask=msk)
        pltpu.sync_copy(zbuf_ref.at[pl.ds(0, nwords)],
                        out_hbm.at[pl.ds(lo, nwords)])

    @pl.when(wid < SC_NW - 1)
    def _():
        assemble(SC_CHUNK)

    @pl.when(wid == SC_NW - 1)
    def _():
        assemble(SC_LAST)


def _run_sc(pk_flat, ik_flat):
    return pl.kernel(
        _sc_body,
        mesh=plsc.VectorSubcoreMesh(core_axis_name="c", subcore_axis_name="s"),
        out_type=jax.ShapeDtypeStruct((VOCAB,), jnp.float32),
        scratch_types=[
            pltpu.VMEM((CAP,), jnp.float32),
            pltpu.VMEM((CAP,), jnp.int32),
            pltpu.VMEM((SC_CHUNK,), jnp.float32),
        ],
        compiler_params=pltpu.CompilerParams(needs_layout_passes=False),
    )(pk_flat, ik_flat)


def kernel(next_token_logits):
    x3 = next_token_logits.reshape(2, ROWS, 128)
    pk, ik, tok = _run_tc(x3)
    probs = _run_sc(pk.reshape(CAP), ik.reshape(CAP))
    return probs.reshape(1, VOCAB), tok.reshape(1)


# speedup vs baseline: 113.3726x; 1.0163x over previous
import jax
import jax.numpy as jnp
from jax.experimental import pallas as pl
from jax.experimental.pallas import tpu as pltpu
from jax.experimental.pallas import tpu_sc as plsc

VOCAB = 215680
ROWS = VOCAB // 128
W_ROW0 = 1184
W_LO = W_ROW0 * 128
WIN_ROWS = ROWS - W_ROW0
PAD_ROWS = 504
BAND_LO = 117
BAND_HI = 64117
EOS_LOC = 93
TOP_K = 50
CAP = 64
NR = 16
MIN_P = 0.05
TOP_P = 0.9
TEMPERATURE = 0.85
PAD_IDX = 0x7FFFFFFF
PAD_BASE = 0x40000000
F32_TINY = float(jnp.finfo(jnp.float32).tiny)


def _threefry_bits(gi_u32):
    x0 = jnp.zeros_like(gi_u32)
    x1 = gi_u32
    ks0 = jnp.uint32(0)
    ks1 = jnp.uint32(1)
    ks2 = jnp.uint32(0x1BD11BDA) ^ ks0 ^ ks1
    rot1 = (13, 15, 26, 6)
    rot2 = (17, 29, 16, 24)
    x0 = x0 + ks0
    x1 = x1 + ks1
    ks = (ks1, ks2, ks0)
    for g in range(5):
        for r in (rot1 if g % 2 == 0 else rot2):
            x0 = x0 + x1
            x1 = (x1 << jnp.uint32(r)) | (x1 >> jnp.uint32(32 - r))
            x1 = x1 ^ x0
        x0 = x0 + ks[g % 3]
        x1 = x1 + ks[(g + 1) % 3] + jnp.uint32(g + 1)
    return x0 ^ x1


def _tc_body(x_ref, pk_ref, ik_ref, tok_ref, key_ref):
    f32 = jnp.float32
    i32 = jnp.int32
    u32 = jnp.uint32
    neg_inf = f32(-jnp.inf)

    c = x_ref[0, W_ROW0:, :]
    u = x_ref[1, W_ROW0:, :]
    cfg = u + f32(2.0) * (c - u)
    ridx = jax.lax.broadcasted_iota(i32, (WIN_ROWS, 128), 0)
    lidx = jax.lax.broadcasted_iota(i32, (WIN_ROWS, 128), 1)
    flat = ridx * 128 + lidx
    valid = ((flat >= BAND_LO) & (flat < BAND_HI)) | (flat == EOS_LOC)
    w = jnp.where(valid, cfg + f32(0.0), neg_inf)
    b = jax.lax.bitcast_convert_type(w, u32)
    key = jnp.where((b >> u32(31)) == u32(1), ~b, b | u32(0x80000000))
    key_ref[0:WIN_ROWS, :] = key
    key_ref[WIN_ROWS:, :] = jnp.zeros((PAD_ROWS - WIN_ROWS, 128), u32)

    def bit_step(i, t):
        sh = u32(30) - u32(2) * i.astype(u32)
        b1 = u32(2) << sh
        b2 = u32(1) << sh
        ka = key_ref[...]
        c1 = jnp.sum((ka >= (t | b1)).astype(i32))
        c2 = jnp.sum((ka >= (t | b2)).astype(i32))
        c3 = jnp.sum((ka >= (t | b1 | b2)).astype(i32))
        hi_ok = c1 >= TOP_K
        t2 = jnp.where(hi_ok, t | b1, t)
        lo_cnt = jnp.where(hi_ok, c3, c2)
        return jnp.where(lo_cnt >= TOP_K, t2 | b2, t2)

    t = jax.lax.fori_loop(0, 16, bit_step, u32(0))

    riota504 = jax.lax.broadcasted_iota(i32, (PAD_ROWS, 128), 0)
    lane1 = jax.lax.broadcasted_iota(i32, (1, 128), 1)
    pks = []
    pis = []
    for r in range(NR):
        ka = key_ref[...]
        alive = ka >= t
        rowidx = jnp.min(jnp.where(alive, riota504, i32(1 << 20)),
                         axis=0, keepdims=True)
        eq = (riota504 == rowidx) & alive
        ka_i = jax.lax.bitcast_convert_type(ka, i32)
        kk = jax.lax.bitcast_convert_type(
            jnp.sum(jnp.where(eq, ka_i, i32(0)), axis=0, keepdims=True), u32)
        live = kk > u32(0)
        pks.append(kk)
        pis.append(jnp.where(live, rowidx * 128 + lane1,
                             i32(PAD_BASE) + r * 128 + lane1))
        key_ref[...] = jnp.where(eq, u32(0), ka)
    psk = jnp.concatenate(pks, axis=0)
    psi = jnp.concatenate(pis, axis=0)

    live = psk > u32(0)
    vb = jnp.where(psk >= u32(0x80000000), psk ^ u32(0x80000000), ~psk)
    psv = jnp.where(live, jax.lax.bitcast_convert_type(vb, f32), neg_inf)

    riota = jax.lax.broadcasted_iota(i32, (NR, 128), 0)
    liota = jax.lax.broadcasted_iota(i32, (NR, 128), 1)
    fr = riota * 128 + liota
    N = NR * 128

    def roll_rows(x, d):
        return jnp.concatenate([x[d:], x[:d]], axis=0)

    k2 = 2
    while k2 <= N:
        dirdesc = (fr & k2) == 0
        j = k2 // 2
        while j >= 1:
            lobit = (fr & j) == 0
            if j < 128:
                up_v = pltpu.roll(psv, 128 - j, axis=1)
                dn_v = pltpu.roll(psv, j, axis=1)
                up_i = pltpu.roll(psi, 128 - j, axis=1)
                dn_i = pltpu.roll(psi, j, axis=1)
            else:
                d = j // 128
                up_v, dn_v = roll_rows(psv, d), roll_rows(psv, NR - d)
                up_i, dn_i = roll_rows(psi, d), roll_rows(psi, NR - d)
            svp = jnp.where(lobit, up_v, dn_v)
            sip = jnp.where(lobit, up_i, dn_i)
            afirst = (psv > svp) | ((psv == svp) & (psi < sip))
            take_self = ((afirst == lobit) == dirdesc)
            psv = jnp.where(take_self, psv, svp)
            psi = jnp.where(take_self, psi, sip)
            j //= 2
        k2 *= 2

    keepcap = fr < CAP
    psv = jnp.where(keepcap, psv, neg_inf)
    psi = jnp.where(keepcap, psi, i32(PAD_BASE) + fr)

    m = jnp.max(psv)
    e = jnp.exp(psv - m)
    z1 = jnp.sum(e)
    p = e / z1
    pmax = f32(1.0) / z1
    keep1 = p >= f32(MIN_P) * pmax
    v1 = jnp.where(keep1, psv, neg_inf)

    e2 = jnp.exp(v1 - m)
    z2 = jnp.sum(e2)
    p2 = e2 / z2
    lane16 = jax.lax.broadcasted_iota(i32, (NR, 128), 1)
    cs = p2
    d = 1
    while d < 128:
        sh = pltpu.roll(cs, d, axis=1)
        cs = cs + jnp.where(lane16 >= d, sh, f32(0.0))
        d *= 2
    row_tot = cs[:, 127:128]
    ex = jnp.concatenate([jnp.zeros((1, 1), f32), row_tot[:NR - 1]], axis=0)
    d = 1
    while d < NR:
        ex = ex + jnp.concatenate(
            [jnp.zeros((d, 1), f32), ex[:NR - d]], axis=0)
        d *= 2
    ci = cs + ex
    csh = pltpu.roll(ci, 1, axis=1)
    exb = ex + jnp.zeros((NR, 128), f32)
    csh = jnp.where(lane16 == 0, exb, csh)
    keep2 = csh <= f32(TOP_P)
    v2 = jnp.where(keep2, v1, neg_inf)

    s = v2 / f32(TEMPERATURE)
    m3 = m / f32(TEMPERATURE)
    e3 = jnp.exp(s - m3)
    z3 = jnp.sum(e3)
    pf = e3 / z3

    gi = jnp.where(psi >= i32(PAD_BASE), i32(0), i32(W_LO) + psi)
    bits = _threefry_bits(jax.lax.bitcast_convert_type(gi, u32))
    fb = (bits >> u32(9)) | u32(0x3F800000)
    frac = jax.lax.bitcast_convert_type(fb, f32) - f32(1.0)
    uu = jnp.maximum(f32(F32_TINY), frac + f32(F32_TINY))
    g = -jnp.log(-jnp.log(uu))
    score = s + g
    msc = jnp.max(score)
    tokv = jnp.min(jnp.where(score == msc, gi, i32(PAD_IDX)))
    tok_ref[0, 0] = tokv

    gvi = jnp.where(psi >= i32(PAD_BASE), i32(PAD_IDX), i32(W_LO) + psi)
    pk_ref[...] = pf[0:1, :]
    ik_ref[...] = gvi[0:1, :]


def _run_tc(x3, interpret=False):
    return pl.pallas_call(
        _tc_body,
        out_shape=(jax.ShapeDtypeStruct((1, 128), jnp.float32),
                   jax.ShapeDtypeStruct((1, 128), jnp.int32),
                   jax.ShapeDtypeStruct((1, 1), jnp.int32)),
        in_specs=[pl.BlockSpec(memory_space=pltpu.VMEM)],
        out_specs=(pl.BlockSpec(memory_space=pltpu.VMEM),
                   pl.BlockSpec(memory_space=pltpu.VMEM),
                   pl.BlockSpec(memory_space=pltpu.SMEM)),
        scratch_shapes=[
            pltpu.VMEM((PAD_ROWS, 128), jnp.uint32),
        ],
        interpret=interpret,
    )(x3)


SC_NW = 32
SC_CHUNK = 6784
SC_LAST = VOCAB - (SC_NW - 1) * SC_CHUNK


def _sc_body(pk_hbm, ik_hbm, out_hbm, pv_ref, iv_ref, zbuf_ref):
    wid = jax.lax.axis_index("s") * 2 + jax.lax.axis_index("c")
    pltpu.sync_copy(pk_hbm, pv_ref)
    pltpu.sync_copy(ik_hbm, iv_ref)
    lo = wid * SC_CHUNK

    def assemble(nwords):
        def zstep(i, carry):
            for q in range(8):
                zbuf_ref[pl.ds(i * 128 + q * 16, 16)] = (
                    jnp.zeros((16,), jnp.float32))
            return carry

        jax.lax.fori_loop(0, nwords // 128, zstep, 0)
        for jj in range(CAP // 16):
            ivv = iv_ref[0, pl.ds(jj * 16, 16)]
            pvv = pv_ref[0, pl.ds(jj * 16, 16)]
            msk = (ivv >= lo) & (ivv < lo + nwords)
            plsc.store_scatter(zbuf_ref, [ivv - lo], pvv, mask=msk)
        pltpu.sync_copy(zbuf_ref.at[pl.ds(0, nwords)],
                        out_hbm.at[pl.ds(lo, nwords)])

    @pl.when(wid < SC_NW - 1)
    def _():
        assemble(SC_CHUNK)

    @pl.when(wid == SC_NW - 1)
    def _():
        assemble(SC_LAST)


def _run_sc(pk_flat, ik_flat):
    return pl.kernel(
        _sc_body,
        mesh=plsc.VectorSubcoreMesh(core_axis_name="c", subcore_axis_name="s"),
        out_type=jax.ShapeDtypeStruct((VOCAB,), jnp.float32),
        scratch_types=[
            pltpu.VMEM((1, 128), jnp.float32),
            pltpu.VMEM((1, 128), jnp.int32),
            pltpu.VMEM((SC_CHUNK,), jnp.float32),
        ],
        compiler_params=pltpu.CompilerParams(needs_layout_passes=False),
    )(pk_flat, ik_flat)


def kernel(next_token_logits):
    x3 = next_token_logits.reshape(2, ROWS, 128)
    pk, ik, tok = _run_tc(x3)
    probs = _run_sc(pk, ik)
    return probs.reshape(1, VOCAB), tok.reshape(1)


# speedup vs baseline: 113.8394x; 1.0041x over previous
import jax
import jax.numpy as jnp
from jax.experimental import pallas as pl
from jax.experimental.pallas import tpu as pltpu
from jax.experimental.pallas import tpu_sc as plsc

VOCAB = 215680
ROWS = VOCAB // 128
W_ROW0 = 1184
W_LO = W_ROW0 * 128
WIN_ROWS = ROWS - W_ROW0
PAD_ROWS = 504
BAND_LO = 117
BAND_HI = 64117
EOS_LOC = 93
TOP_K = 50
CAP = 64
NR = 16
MIN_P = 0.05
TOP_P = 0.9
TEMPERATURE = 0.85
PAD_IDX = 0x7FFFFFFF
PAD_BASE = 0x40000000
F32_TINY = float(jnp.finfo(jnp.float32).tiny)


def _threefry_bits(gi_u32):
    x0 = jnp.zeros_like(gi_u32)
    x1 = gi_u32
    ks0 = jnp.uint32(0)
    ks1 = jnp.uint32(1)
    ks2 = jnp.uint32(0x1BD11BDA) ^ ks0 ^ ks1
    rot1 = (13, 15, 26, 6)
    rot2 = (17, 29, 16, 24)
    x0 = x0 + ks0
    x1 = x1 + ks1
    ks = (ks1, ks2, ks0)
    for g in range(5):
        for r in (rot1 if g % 2 == 0 else rot2):
            x0 = x0 + x1
            x1 = (x1 << jnp.uint32(r)) | (x1 >> jnp.uint32(32 - r))
            x1 = x1 ^ x0
        x0 = x0 + ks[g % 3]
        x1 = x1 + ks[(g + 1) % 3] + jnp.uint32(g + 1)
    return x0 ^ x1


def _tc_body(x_ref, pk_ref, ik_ref, tok_ref, key_ref):
    f32 = jnp.float32
    i32 = jnp.int32
    u32 = jnp.uint32
    neg_inf = f32(-jnp.inf)

    c = x_ref[0, W_ROW0:, :]
    u = x_ref[1, W_ROW0:, :]
    cfg = u + f32(2.0) * (c - u)
    ridx = jax.lax.broadcasted_iota(i32, (WIN_ROWS, 128), 0)
    lidx = jax.lax.broadcasted_iota(i32, (WIN_ROWS, 128), 1)
    flat = ridx * 128 + lidx
    valid = ((flat >= BAND_LO) & (flat < BAND_HI)) | (flat == EOS_LOC)
    w = jnp.where(valid, cfg + f32(0.0), neg_inf)
    b = jax.lax.bitcast_convert_type(w, u32)
    key = jnp.where((b >> u32(31)) == u32(1), ~b, b | u32(0x80000000))
    key_ref[0:WIN_ROWS, :] = key
    key_ref[WIN_ROWS:, :] = jnp.zeros((PAD_ROWS - WIN_ROWS, 128), u32)

    def bit_step(i, t):
        sh = u32(30) - u32(2) * i.astype(u32)
        b1 = u32(2) << sh
        b2 = u32(1) << sh
        ka = key_ref[...]
        c1 = jnp.sum((ka >= (t | b1)).astype(i32))
        c2 = jnp.sum((ka >= (t | b2)).astype(i32))
        c3 = jnp.sum((ka >= (t | b1 | b2)).astype(i32))
        hi_ok = c1 >= TOP_K
        t2 = jnp.where(hi_ok, t | b1, t)
        lo_cnt = jnp.where(hi_ok, c3, c2)
        return jnp.where(lo_cnt >= TOP_K, t2 | b2, t2)

    t = jax.lax.fori_loop(0, 16, bit_step, u32(0))

    riota504 = jax.lax.broadcasted_iota(i32, (PAD_ROWS, 128), 0)
    lane1 = jax.lax.broadcasted_iota(i32, (1, 128), 1)
    pks = []
    pis = []
    for r in range(NR):
        ka = key_ref[...]
        alive = ka >= t
        rowidx = jnp.min(jnp.where(alive, riota504, i32(1 << 20)),
                         axis=0, keepdims=True)
        eq = (riota504 == rowidx) & alive
        ka_i = jax.lax.bitcast_convert_type(ka, i32)
        kk = jax.lax.bitcast_convert_type(
            jnp.sum(jnp.where(eq, ka_i, i32(0)), axis=0, keepdims=True), u32)
        live = kk > u32(0)
        pks.append(kk)
        pis.append(jnp.where(live, rowidx * 128 + lane1,
                             i32(PAD_BASE) + r * 128 + lane1))
        key_ref[...] = jnp.where(eq, u32(0), ka)
    psk = jnp.concatenate(pks, axis=0)
    psi = jnp.concatenate(pis, axis=0)

    live = psk > u32(0)
    vb = jnp.where(psk >= u32(0x80000000), psk ^ u32(0x80000000), ~psk)
    psv = jnp.where(live, jax.lax.bitcast_convert_type(vb, f32), neg_inf)

    riota = jax.lax.broadcasted_iota(i32, (NR, 128), 0)
    liota = jax.lax.broadcasted_iota(i32, (NR, 128), 1)
    fr = riota * 128 + liota
    N = NR * 128

    def roll_rows(x, d):
        return jnp.concatenate([x[d:], x[:d]], axis=0)

    k2 = 2
    while k2 <= N:
        dirdesc = (fr & k2) == 0
        j = k2 // 2
        while j >= 1:
            lobit = (fr & j) == 0
            if j < 128:
                up_v = pltpu.roll(psv, 128 - j, axis=1)
                dn_v = pltpu.roll(psv, j, axis=1)
                up_i = pltpu.roll(psi, 128 - j, axis=1)
                dn_i = pltpu.roll(psi, j, axis=1)
            else:
                d = j // 128
                up_v, dn_v = roll_rows(psv, d), roll_rows(psv, NR - d)
                up_i, dn_i = roll_rows(psi, d), roll_rows(psi, NR - d)
            svp = jnp.where(lobit, up_v, dn_v)
            sip = jnp.where(lobit, up_i, dn_i)
            afirst = (psv > svp) | ((psv == svp) & (psi < sip))
            take_self = ((afirst == lobit) == dirdesc)
            psv = jnp.where(take_self, psv, svp)
            psi = jnp.where(take_self, psi, sip)
            j //= 2
        k2 *= 2

    lane1i = jax.lax.broadcasted_iota(i32, (1, 128), 1)
    keepcap = lane1i < CAP
    pv0 = jnp.where(keepcap, psv[0:1, :], neg_inf)
    pi0 = jnp.where(keepcap, psi[0:1, :], i32(PAD_BASE) + lane1i)

    m = jnp.max(pv0)
    e = jnp.exp(pv0 - m)
    z1 = jnp.sum(e)
    p = e / z1
    pmax = f32(1.0) / z1
    keep1 = p >= f32(MIN_P) * pmax
    v1 = jnp.where(keep1, pv0, neg_inf)

    e2 = jnp.exp(v1 - m)
    z2 = jnp.sum(e2)
    p2 = e2 / z2
    cs = p2
    d = 1
    while d < 128:
        sh = pltpu.roll(cs, d, axis=1)
        cs = cs + jnp.where(lane1i >= d, sh, f32(0.0))
        d *= 2
    csh = pltpu.roll(cs, 1, axis=1)
    csh = jnp.where(lane1i == 0, f32(0.0), csh)
    keep2 = csh <= f32(TOP_P)
    v2 = jnp.where(keep2, v1, neg_inf)

    s = v2 / f32(TEMPERATURE)
    m3 = m / f32(TEMPERATURE)
    e3 = jnp.exp(s - m3)
    z3 = jnp.sum(e3)
    pf = e3 / z3

    gi = jnp.where(pi0 >= i32(PAD_BASE), i32(0), i32(W_LO) + pi0)
    bits = _threefry_bits(jax.lax.bitcast_convert_type(gi, u32))
    fb = (bits >> u32(9)) | u32(0x3F800000)
    frac = jax.lax.bitcast_convert_type(fb, f32) - f32(1.0)
    uu = jnp.maximum(f32(F32_TINY), frac + f32(F32_TINY))
    g = -jnp.log(-jnp.log(uu))
    score = s + g
    msc = jnp.max(score)
    tokv = jnp.min(jnp.where(score == msc, gi, i32(PAD_IDX)))
    tok_ref[0, 0] = tokv

    gvi = jnp.where(pi0 >= i32(PAD_BASE), i32(PAD_IDX), i32(W_LO) + pi0)
    pk_ref[...] = pf
    ik_ref[...] = gvi


def _run_tc(x3, interpret=False):
    return pl.pallas_call(
        _tc_body,
        out_shape=(jax.ShapeDtypeStruct((1, 128), jnp.float32),
                   jax.ShapeDtypeStruct((1, 128), jnp.int32),
                   jax.ShapeDtypeStruct((1, 1), jnp.int32)),
        in_specs=[pl.BlockSpec(memory_space=pltpu.VMEM)],
        out_specs=(pl.BlockSpec(memory_space=pltpu.VMEM),
                   pl.BlockSpec(memory_space=pltpu.VMEM),
                   pl.BlockSpec(memory_space=pltpu.SMEM)),
        scratch_shapes=[
            pltpu.VMEM((PAD_ROWS, 128), jnp.uint32),
        ],
        compiler_params=None if interpret else pltpu.CompilerParams(
            allow_input_fusion=[True]),
        interpret=interpret,
    )(x3)


SC_NW = 32
SC_CHUNK = 6784
SC_LAST = VOCAB - (SC_NW - 1) * SC_CHUNK


def _sc_body(pk_hbm, ik_hbm, out_hbm, pv_ref, iv_ref, zbuf_ref):
    wid = jax.lax.axis_index("s") * 2 + jax.lax.axis_index("c")
    pltpu.sync_copy(pk_hbm, pv_ref)
    pltpu.sync_copy(ik_hbm, iv_ref)
    lo = wid * SC_CHUNK

    def assemble(nwords):
        def zstep(i, carry):
            for q in range(8):
                zbuf_ref[pl.ds(i * 128 + q * 16, 16)] = (
                    jnp.zeros((16,), jnp.float32))
            return carry

        jax.lax.fori_loop(0, nwords // 128, zstep, 0)
        for jj in range(CAP // 16):
            ivv = iv_ref[0, pl.ds(jj * 16, 16)]
            pvv = pv_ref[0, pl.ds(jj * 16, 16)]
            msk = (ivv >= lo) & (ivv < lo + nwords)
            plsc.store_scatter(zbuf_ref, [ivv - lo], pvv, mask=msk)
        pltpu.sync_copy(zbuf_ref.at[pl.ds(0, nwords)],
                        out_hbm.at[pl.ds(lo, nwords)])

    @pl.when(wid < SC_NW - 1)
    def _():
        assemble(SC_CHUNK)

    @pl.when(wid == SC_NW - 1)
    def _():
        assemble(SC_LAST)


def _run_sc(pk_flat, ik_flat):
    return pl.kernel(
        _sc_body,
        mesh=plsc.VectorSubcoreMesh(core_axis_name="c", subcore_axis_name="s"),
        out_type=jax.ShapeDtypeStruct((VOCAB,), jnp.float32),
        scratch_types=[
            pltpu.VMEM((1, 128), jnp.float32),
            pltpu.VMEM((1, 128), jnp.int32),
            pltpu.VMEM((SC_CHUNK,), jnp.float32),
        ],
        compiler_params=pltpu.CompilerParams(needs_layout_passes=False),
    )(pk_flat, ik_flat)


def kernel(next_token_logits):
    x3 = next_token_logits.reshape(2, ROWS, 128)
    pk, ik, tok = _run_tc(x3)
    probs = _run_sc(pk, ik)
    return probs.reshape(1, VOCAB), tok.reshape(1)


# speedup vs baseline: 114.0369x; 1.0017x over previous
import jax
import jax.numpy as jnp
from jax.experimental import pallas as pl
from jax.experimental.pallas import tpu as pltpu
from jax.experimental.pallas import tpu_sc as plsc

VOCAB = 215680
ROWS = VOCAB // 128
W_ROW0 = 1184
W_LO = W_ROW0 * 128
WIN_ROWS = ROWS - W_ROW0
PAD_ROWS = 504
BAND_LO = 117
BAND_HI = 64117
EOS_LOC = 93
TOP_K = 50
CAP = 64
NR = 16
MIN_P = 0.05
TOP_P = 0.9
TEMPERATURE = 0.85
PAD_IDX = 0x7FFFFFFF
PAD_BASE = 0x40000000
F32_TINY = float(jnp.finfo(jnp.float32).tiny)


def _threefry_bits(gi_u32):
    x0 = jnp.zeros_like(gi_u32)
    x1 = gi_u32
    ks0 = jnp.uint32(0)
    ks1 = jnp.uint32(1)
    ks2 = jnp.uint32(0x1BD11BDA) ^ ks0 ^ ks1
    rot1 = (13, 15, 26, 6)
    rot2 = (17, 29, 16, 24)
    x0 = x0 + ks0
    x1 = x1 + ks1
    ks = (ks1, ks2, ks0)
    for g in range(5):
        for r in (rot1 if g % 2 == 0 else rot2):
            x0 = x0 + x1
            x1 = (x1 << jnp.uint32(r)) | (x1 >> jnp.uint32(32 - r))
            x1 = x1 ^ x0
        x0 = x0 + ks[g % 3]
        x1 = x1 + ks[(g + 1) % 3] + jnp.uint32(g + 1)
    return x0 ^ x1


def _tc_body(x_ref, pk_ref, ik_ref, tok_ref, key_ref):
    f32 = jnp.float32
    i32 = jnp.int32
    u32 = jnp.uint32
    neg_inf = f32(-jnp.inf)

    c = x_ref[0, W_ROW0:, :]
    u = x_ref[1, W_ROW0:, :]
    cfg = u + f32(2.0) * (c - u)
    ridx = jax.lax.broadcasted_iota(i32, (WIN_ROWS, 128), 0)
    lidx = jax.lax.broadcasted_iota(i32, (WIN_ROWS, 128), 1)
    flat = ridx * 128 + lidx
    valid = ((flat >= BAND_LO) & (flat < BAND_HI)) | (flat == EOS_LOC)
    w = jnp.where(valid, cfg + f32(0.0), neg_inf)
    b = jax.lax.bitcast_convert_type(w, u32)
    key = jnp.where((b >> u32(31)) == u32(1), ~b, b | u32(0x80000000))
    key_ref[0:WIN_ROWS, :] = key
    key_ref[WIN_ROWS:, :] = jnp.zeros((PAD_ROWS - WIN_ROWS, 128), u32)

    def bit_step(i, t):
        sh = u32(30) - u32(2) * i.astype(u32)
        b1 = u32(2) << sh
        b2 = u32(1) << sh
        ka = key_ref[...]
        c1 = jnp.sum((ka >= (t | b1)).astype(i32))
        c2 = jnp.sum((ka >= (t | b2)).astype(i32))
        c3 = jnp.sum((ka >= (t | b1 | b2)).astype(i32))
        hi_ok = c1 >= TOP_K
        t2 = jnp.where(hi_ok, t | b1, t)
        lo_cnt = jnp.where(hi_ok, c3, c2)
        return jnp.where(lo_cnt >= TOP_K, t2 | b2, t2)

    t = jax.lax.fori_loop(0, 16, bit_step, u32(0))

    riota504 = jax.lax.broadcasted_iota(i32, (PAD_ROWS, 128), 0)
    lane1 = jax.lax.broadcasted_iota(i32, (1, 128), 1)
    pks = []
    pis = []
    for r in range(NR):
        ka = key_ref[...]
        alive = ka >= t
        rowidx = jnp.min(jnp.where(alive, riota504, i32(1 << 20)),
                         axis=0, keepdims=True)
        eq = (riota504 == rowidx) & alive
        ka_i = jax.lax.bitcast_convert_type(ka, i32)
        kk = jax.lax.bitcast_convert_type(
            jnp.sum(jnp.where(eq, ka_i, i32(0)), axis=0, keepdims=True), u32)
        live = kk > u32(0)
        pks.append(kk)
        pis.append(jnp.where(live, rowidx * 128 + lane1,
                             i32(PAD_BASE) + r * 128 + lane1))
        key_ref[...] = jnp.where(eq, u32(0), ka)
    psk = jnp.concatenate(pks, axis=0)
    psi = jnp.concatenate(pis, axis=0)

    live = psk > u32(0)
    vb = jnp.where(psk >= u32(0x80000000), psk ^ u32(0x80000000), ~psk)
    psv = jnp.where(live, jax.lax.bitcast_convert_type(vb, f32), neg_inf)

    riota = jax.lax.broadcasted_iota(i32, (NR, 128), 0)
    liota = jax.lax.broadcasted_iota(i32, (NR, 128), 1)
    fr = riota * 128 + liota
    N = NR * 128

    def roll_rows(x, d):
        return jnp.concatenate([x[d:], x[:d]], axis=0)

    k2 = 2
    while k2 <= N:
        dirdesc = (fr & k2) == 0
        j = k2 // 2
        while j >= 1:
            lobit = (fr & j) == 0
            if j < 128:
                up_v = pltpu.roll(psv, 128 - j, axis=1)
                dn_v = pltpu.roll(psv, j, axis=1)
                up_i = pltpu.roll(psi, 128 - j, axis=1)
                dn_i = pltpu.roll(psi, j, axis=1)
            else:
                d = j // 128
                up_v, dn_v = roll_rows(psv, d), roll_rows(psv, NR - d)
                up_i, dn_i = roll_rows(psi, d), roll_rows(psi, NR - d)
            svp = jnp.where(lobit, up_v, dn_v)
            sip = jnp.where(lobit, up_i, dn_i)
            afirst = (psv > svp) | ((psv == svp) & (psi < sip))
            take_self = ((afirst == lobit) == dirdesc)
            psv = jnp.where(take_self, psv, svp)
            psi = jnp.where(take_self, psi, sip)
            j //= 2
        k2 *= 2

    lane1i = jax.lax.broadcasted_iota(i32, (1, 128), 1)
    keepcap = lane1i < CAP
    pv0 = jnp.where(keepcap, psv[0:1, :], neg_inf)
    pi0 = jnp.where(keepcap, psi[0:1, :], i32(PAD_BASE) + lane1i)

    m = jnp.max(pv0)
    e = jnp.exp(pv0 - m)
    z1 = jnp.sum(e)
    p = e / z1
    pmax = f32(1.0) / z1
    keep1 = p >= f32(MIN_P) * pmax
    v1 = jnp.where(keep1, pv0, neg_inf)

    e2 = jnp.exp(v1 - m)
    z2 = jnp.sum(e2)
    p2 = e2 / z2
    cs = p2
    d = 1
    while d < 128:
        sh = pltpu.roll(cs, d, axis=1)
        cs = cs + jnp.where(lane1i >= d, sh, f32(0.0))
        d *= 2
    csh = pltpu.roll(cs, 1, axis=1)
    csh = jnp.where(lane1i == 0, f32(0.0), csh)
    keep2 = csh <= f32(TOP_P)
    v2 = jnp.where(keep2, v1, neg_inf)

    s = v2 / f32(TEMPERATURE)
    m3 = m / f32(TEMPERATURE)
    e3 = jnp.exp(s - m3)
    z3 = jnp.sum(e3)
    pf = e3 / z3

    gi = jnp.where(pi0 >= i32(PAD_BASE), i32(0), i32(W_LO) + pi0)
    bits = _threefry_bits(jax.lax.bitcast_convert_type(gi, u32))
    fb = (bits >> u32(9)) | u32(0x3F800000)
    frac = jax.lax.bitcast_convert_type(fb, f32) - f32(1.0)
    uu = jnp.maximum(f32(F32_TINY), frac + f32(F32_TINY))
    g = -jnp.log(-jnp.log(uu))
    score = s + g
    msc = jnp.max(score)
    tokv = jnp.min(jnp.where(score == msc, gi, i32(PAD_IDX)))
    tok_ref[0, 0] = tokv

    gvi = jnp.where(pi0 >= i32(PAD_BASE), i32(PAD_IDX), i32(W_LO) + pi0)
    pk_ref[...] = pf
    ik_ref[...] = gvi


def _run_tc(x3):
    return pl.pallas_call(
        _tc_body,
        out_shape=(jax.ShapeDtypeStruct((1, 128), jnp.float32),
                   jax.ShapeDtypeStruct((1, 128), jnp.int32),
                   jax.ShapeDtypeStruct((1, 1), jnp.int32)),
        in_specs=[pl.BlockSpec(memory_space=pltpu.VMEM)],
        out_specs=(pl.BlockSpec(memory_space=pltpu.VMEM),
                   pl.BlockSpec(memory_space=pltpu.VMEM),
                   pl.BlockSpec(memory_space=pltpu.SMEM)),
        scratch_shapes=[
            pltpu.VMEM((PAD_ROWS, 128), jnp.uint32),
        ],
        compiler_params=pltpu.CompilerParams(allow_input_fusion=[True]),
    )(x3)


SC_NW = 32
SC_CHUNK = 6784
SC_LAST = VOCAB - (SC_NW - 1) * SC_CHUNK


def _sc_body(pk_hbm, ik_hbm, out_hbm, pv_ref, iv_ref, zbuf_ref):
    wid = jax.lax.axis_index("s") * 2 + jax.lax.axis_index("c")
    pltpu.sync_copy(pk_hbm, pv_ref)
    pltpu.sync_copy(ik_hbm, iv_ref)
    lo = wid * SC_CHUNK

    def assemble(nwords):
        def zstep(i, carry):
            for q in range(8):
                zbuf_ref[pl.ds(i * 128 + q * 16, 16)] = (
                    jnp.zeros((16,), jnp.float32))
            return carry

        jax.lax.fori_loop(0, nwords // 128, zstep, 0)
        for jj in range(CAP // 16):
            ivv = iv_ref[0, pl.ds(jj * 16, 16)]
            pvv = pv_ref[0, pl.ds(jj * 16, 16)]
            msk = (ivv >= lo) & (ivv < lo + nwords)
            plsc.store_scatter(zbuf_ref, [ivv - lo], pvv, mask=msk)
        pltpu.sync_copy(zbuf_ref.at[pl.ds(0, nwords)],
                        out_hbm.at[pl.ds(lo, nwords)])

    @pl.when(wid < SC_NW - 1)
    def _():
        assemble(SC_CHUNK)

    @pl.when(wid == SC_NW - 1)
    def _():
        assemble(SC_LAST)


def _run_sc(pk_flat, ik_flat):
    return pl.kernel(
        _sc_body,
        mesh=plsc.VectorSubcoreMesh(core_axis_name="c", subcore_axis_name="s"),
        out_type=jax.ShapeDtypeStruct((VOCAB,), jnp.float32),
        scratch_types=[
            pltpu.VMEM((1, 128), jnp.float32),
            pltpu.VMEM((1, 128), jnp.int32),
            pltpu.VMEM((SC_CHUNK,), jnp.float32),
        ],
        compiler_params=pltpu.CompilerParams(needs_layout_passes=False),
    )(pk_flat, ik_flat)


def kernel(next_token_logits):
    x3 = next_token_logits.reshape(2, ROWS, 128)
    pk, ik, tok = _run_tc(x3)
    probs = _run_sc(pk, ik)
    return probs.reshape(1, VOCAB), tok.reshape(1)


# speedup vs baseline: 116.7250x; 1.0236x over previous
import jax
import jax.numpy as jnp
from jax.experimental import pallas as pl
from jax.experimental.pallas import tpu as pltpu
from jax.experimental.pallas import tpu_sc as plsc

VOCAB = 215680
ROWS = VOCAB // 128
W_ROW0 = 1184
W_LO = W_ROW0 * 128
WIN_ROWS = ROWS - W_ROW0
PAD_ROWS = 504
BAND_LO = 117
BAND_HI = 64117
EOS_LOC = 93
TOP_K = 50
CAP = 64
NR = 16
MIN_P = 0.05
TOP_P = 0.9
TEMPERATURE = 0.85
PAD_IDX = 0x7FFFFFFF
PAD_BASE = 0x40000000
F32_TINY = float(jnp.finfo(jnp.float32).tiny)


def _threefry_bits(gi_u32):
    x0 = jnp.zeros_like(gi_u32)
    x1 = gi_u32
    ks0 = jnp.uint32(0)
    ks1 = jnp.uint32(1)
    ks2 = jnp.uint32(0x1BD11BDA) ^ ks0 ^ ks1
    rot1 = (13, 15, 26, 6)
    rot2 = (17, 29, 16, 24)
    x0 = x0 + ks0
    x1 = x1 + ks1
    ks = (ks1, ks2, ks0)
    for g in range(5):
        for r in (rot1 if g % 2 == 0 else rot2):
            x0 = x0 + x1
            x1 = (x1 << jnp.uint32(r)) | (x1 >> jnp.uint32(32 - r))
            x1 = x1 ^ x0
        x0 = x0 + ks[g % 3]
        x1 = x1 + ks[(g + 1) % 3] + jnp.uint32(g + 1)
    return x0 ^ x1


def _tc_body(x_ref, pk_ref, ik_ref, tok_ref, key_ref, psk_ref, psi_ref):
    f32 = jnp.float32
    i32 = jnp.int32
    u32 = jnp.uint32
    neg_inf = f32(-jnp.inf)

    c = x_ref[0, W_ROW0:, :]
    u = x_ref[1, W_ROW0:, :]
    cfg = u + f32(2.0) * (c - u)
    ridx = jax.lax.broadcasted_iota(i32, (WIN_ROWS, 128), 0)
    lidx = jax.lax.broadcasted_iota(i32, (WIN_ROWS, 128), 1)
    flat = ridx * 128 + lidx
    valid = ((flat >= BAND_LO) & (flat < BAND_HI)) | (flat == EOS_LOC)
    w = jnp.where(valid, cfg + f32(0.0), neg_inf)
    b = jax.lax.bitcast_convert_type(w, u32)
    key = jnp.where((b >> u32(31)) == u32(1), ~b, b | u32(0x80000000))
    key_ref[0:WIN_ROWS, :] = key
    key_ref[WIN_ROWS:, :] = jnp.zeros((PAD_ROWS - WIN_ROWS, 128), u32)

    def bit_step(i, t):
        sh = u32(30) - u32(2) * i.astype(u32)
        b1 = u32(2) << sh
        b2 = u32(1) << sh
        ka = key_ref[...]
        c1 = jnp.sum((ka >= (t | b1)).astype(i32))
        c2 = jnp.sum((ka >= (t | b2)).astype(i32))
        c3 = jnp.sum((ka >= (t | b1 | b2)).astype(i32))
        hi_ok = c1 >= TOP_K
        t2 = jnp.where(hi_ok, t | b1, t)
        lo_cnt = jnp.where(hi_ok, c3, c2)
        return jnp.where(lo_cnt >= TOP_K, t2 | b2, t2)

    t = jax.lax.fori_loop(0, 16, bit_step, u32(0))

    riota504 = jax.lax.broadcasted_iota(i32, (PAD_ROWS, 128), 0)
    lane1 = jax.lax.broadcasted_iota(i32, (1, 128), 1)
    n_surv = jnp.sum((key_ref[...] >= t).astype(i32))

    def round_r(r):
        ka = key_ref[...]
        alive = ka >= t
        rowidx = jnp.min(jnp.where(alive, riota504, i32(1 << 20)),
                         axis=0, keepdims=True)
        eq = (riota504 == rowidx) & alive
        ka_i = jax.lax.bitcast_convert_type(ka, i32)
        kk = jax.lax.bitcast_convert_type(
            jnp.sum(jnp.where(eq, ka_i, i32(0)), axis=0, keepdims=True), u32)
        live = kk > u32(0)
        pidx = jnp.where(live, rowidx * 128 + lane1,
                         i32(PAD_BASE) + r * 128 + lane1)
        key_ref[...] = jnp.where(eq, u32(0), ka)
        psk_ref[r:r + 1, :] = kk
        psi_ref[r:r + 1, :] = pidx
        return jnp.sum(live.astype(i32))

    got = i32(0)
    for r in range(4):
        got = got + round_r(r)
    riota16 = jax.lax.broadcasted_iota(i32, (NR, 128), 0)
    liota16 = jax.lax.broadcasted_iota(i32, (NR, 128), 1)
    psk_ref[4:NR, :] = jnp.zeros((NR - 4, 128), u32)
    psi_ref[4:NR, :] = (i32(PAD_BASE) + riota16 * 128 + liota16)[4:NR, :]

    @pl.when(got < n_surv)
    def _():
        for r in range(4, NR):
            round_r(r)

    psk = psk_ref[...]
    psi = psi_ref[...]

    live = psk > u32(0)
    vb = jnp.where(psk >= u32(0x80000000), psk ^ u32(0x80000000), ~psk)
    psv = jnp.where(live, jax.lax.bitcast_convert_type(vb, f32), neg_inf)

    riota = jax.lax.broadcasted_iota(i32, (NR, 128), 0)
    liota = jax.lax.broadcasted_iota(i32, (NR, 128), 1)
    fr = riota * 128 + liota
    N = NR * 128

    def roll_rows(x, d):
        return jnp.concatenate([x[d:], x[:d]], axis=0)

    k2 = 2
    while k2 <= N:
        dirdesc = (fr & k2) == 0
        j = k2 // 2
        while j >= 1:
            lobit = (fr & j) == 0
            if j < 128:
                up_v = pltpu.roll(psv, 128 - j, axis=1)
                dn_v = pltpu.roll(psv, j, axis=1)
                up_i = pltpu.roll(psi, 128 - j, axis=1)
                dn_i = pltpu.roll(psi, j, axis=1)
            else:
                d = j // 128
                up_v, dn_v = roll_rows(psv, d), roll_rows(psv, NR - d)
                up_i, dn_i = roll_rows(psi, d), roll_rows(psi, NR - d)
            svp = jnp.where(lobit, up_v, dn_v)
            sip = jnp.where(lobit, up_i, dn_i)
            afirst = (psv > svp) | ((psv == svp) & (psi < sip))
            take_self = ((afirst == lobit) == dirdesc)
            psv = jnp.where(take_self, psv, svp)
            psi = jnp.where(take_self, psi, sip)
            j //= 2
        k2 *= 2

    lane1i = jax.lax.broadcasted_iota(i32, (1, 128), 1)
    keepcap = lane1i < CAP
    pv0 = jnp.where(keepcap, psv[0:1, :], neg_inf)
    pi0 = jnp.where(keepcap, psi[0:1, :], i32(PAD_BASE) + lane1i)

    m = jnp.max(pv0)
    e = jnp.exp(pv0 - m)
    z1 = jnp.sum(e)
    p = e / z1
    pmax = f32(1.0) / z1
    keep1 = p >= f32(MIN_P) * pmax
    v1 = jnp.where(keep1, pv0, neg_inf)

    e2 = jnp.exp(v1 - m)
    z2 = jnp.sum(e2)
    p2 = e2 / z2
    cs = p2
    d = 1
    while d < 128:
        sh = pltpu.roll(cs, d, axis=1)
        cs = cs + jnp.where(lane1i >= d, sh, f32(0.0))
        d *= 2
    csh = pltpu.roll(cs, 1, axis=1)
    csh = jnp.where(lane1i == 0, f32(0.0), csh)
    keep2 = csh <= f32(TOP_P)
    v2 = jnp.where(keep2, v1, neg_inf)

    s = v2 / f32(TEMPERATURE)
    m3 = m / f32(TEMPERATURE)
    e3 = jnp.exp(s - m3)
    z3 = jnp.sum(e3)
    pf = e3 / z3

    gi = jnp.where(pi0 >= i32(PAD_BASE), i32(0), i32(W_LO) + pi0)
    bits = _threefry_bits(jax.lax.bitcast_convert_type(gi, u32))
    fb = (bits >> u32(9)) | u32(0x3F800000)
    frac = jax.lax.bitcast_convert_type(fb, f32) - f32(1.0)
    uu = jnp.maximum(f32(F32_TINY), frac + f32(F32_TINY))
    g = -jnp.log(-jnp.log(uu))
    score = s + g
    msc = jnp.max(score)
    tokv = jnp.min(jnp.where(score == msc, gi, i32(PAD_IDX)))
    tok_ref[0, 0] = tokv

    gvi = jnp.where(pi0 >= i32(PAD_BASE), i32(PAD_IDX), i32(W_LO) + pi0)
    pk_ref[...] = pf
    ik_ref[...] = gvi


def _run_tc(x3):
    return pl.pallas_call(
        _tc_body,
        out_shape=(jax.ShapeDtypeStruct((1, 128), jnp.float32),
                   jax.ShapeDtypeStruct((1, 128), jnp.int32),
                   jax.ShapeDtypeStruct((1, 1), jnp.int32)),
        in_specs=[pl.BlockSpec(memory_space=pltpu.VMEM)],
        out_specs=(pl.BlockSpec(memory_space=pltpu.VMEM),
                   pl.BlockSpec(memory_space=pltpu.VMEM),
                   pl.BlockSpec(memory_space=pltpu.SMEM)),
        scratch_shapes=[
            pltpu.VMEM((PAD_ROWS, 128), jnp.uint32),
            pltpu.VMEM((NR, 128), jnp.uint32),
            pltpu.VMEM((NR, 128), jnp.int32),
        ],
        compiler_params=pltpu.CompilerParams(allow_input_fusion=[True]),
    )(x3)


SC_NW = 32
SC_CHUNK = 6784
SC_LAST = VOCAB - (SC_NW - 1) * SC_CHUNK


def _sc_body(pk_hbm, ik_hbm, out_hbm, pv_ref, iv_ref, zbuf_ref):
    wid = jax.lax.axis_index("s") * 2 + jax.lax.axis_index("c")
    pltpu.sync_copy(pk_hbm, pv_ref)
    pltpu.sync_copy(ik_hbm, iv_ref)
    lo = wid * SC_CHUNK

    def assemble(nwords):
        def zstep(i, carry):
            for q in range(8):
                zbuf_ref[pl.ds(i * 128 + q * 16, 16)] = (
                    jnp.zeros((16,), jnp.float32))
            return carry

        jax.lax.fori_loop(0, nwords // 128, zstep, 0)
        for jj in range(CAP // 16):
            ivv = iv_ref[0, pl.ds(jj * 16, 16)]
            pvv = pv_ref[0, pl.ds(jj * 16, 16)]
            msk = (ivv >= lo) & (ivv < lo + nwords)
            plsc.store_scatter(zbuf_ref, [ivv - lo], pvv, mask=msk)
        pltpu.sync_copy(zbuf_ref.at[pl.ds(0, nwords)],
                        out_hbm.at[pl.ds(lo, nwords)])

    @pl.when(wid < SC_NW - 1)
    def _():
        assemble(SC_CHUNK)

    @pl.when(wid == SC_NW - 1)
    def _():
        assemble(SC_LAST)


def _run_sc(pk_flat, ik_flat):
    return pl.kernel(
        _sc_body,
        mesh=plsc.VectorSubcoreMesh(core_axis_name="c", subcore_axis_name="s"),
        out_type=jax.ShapeDtypeStruct((VOCAB,), jnp.float32),
        scratch_types=[
            pltpu.VMEM((1, 128), jnp.float32),
            pltpu.VMEM((1, 128), jnp.int32),
            pltpu.VMEM((SC_CHUNK,), jnp.float32),
        ],
        compiler_params=pltpu.CompilerParams(needs_layout_passes=False),
    )(pk_flat, ik_flat)


def kernel(next_token_logits):
    x3 = next_token_logits.reshape(2, ROWS, 128)
    pk, ik, tok = _run_tc(x3)
    probs = _run_sc(pk, ik)
    return probs.reshape(1, VOCAB), tok.reshape(1)
